# baseline (device time: 83209 ns/iter reference)
import jax
import jax.numpy as jnp
from jax import lax
from jax.experimental import pallas as pl
from jax.experimental.pallas import tpu as pltpu

N_DEV = 8
M = 4096
M_PER = 512
N = 2048

G0 = (0, 640)
G1 = (640, 1280)
G2 = (1280, 2048)


def kernel(x, w_mat, scale_x, scale_w):
    def body(x_ref, w_ref, sx_ref, sw_ref, out_ref,
             acc0, acc1, acc2, rcv0, rcv1, rcv2,
             ss0, rs0, ss1, rs1, ss2, rs2):
        p = lax.axis_index("i")
        zb = p >> 2
        yb = (p >> 1) & 1
        xb = (p ^ (p >> 1)) & 1

        def block_dot(w_g, c_start, n_chunks):
            xs = x_ref[pl.ds(c_start * M_PER, n_chunks * M_PER), :]
            r = lax.dot_general(
                xs, w_g, (((1,), (0,)), ((), ())),
                preferred_element_type=jnp.int32,
            ).astype(jnp.bfloat16)
            return r.reshape(n_chunks, M_PER, w_g.shape[1])

        w0 = w_ref[:, G0[0]:G0[1]]
        w1 = w_ref[:, G1[0]:G1[1]]
        w2 = w_ref[:, G2[0]:G2[1]]

        def side_x(t):
            return [jnp.where(t == 0, a, b)
                    for a, b in zip((0, 3, 4, 7), (1, 2, 5, 6))]

        def rdma(acc, rcv, ssem, rsem, a_start, nblk, r_slot, sem, dev):
            return pltpu.make_async_remote_copy(
                src_ref=acc.at[pl.ds(a_start, nblk)],
                dst_ref=rcv.at[pl.ds(r_slot, nblk)],
                send_sem=ssem.at[sem],
                recv_sem=rsem.at[sem],
                device_id=(dev,),
                device_id_type=pl.DeviceIdType.MESH,
            )

        barrier_sem = pltpu.get_barrier_semaphore()
        for nbr in (p ^ 1, p ^ 3, p ^ 4):
            pl.semaphore_signal(
                barrier_sem, inc=1,
                device_id=(nbr,), device_id_type=pl.DeviceIdType.MESH,
            )
        pl.semaphore_wait(barrier_sem, 3)

        acc0[pl.ds(4 * (1 - zb), 4)] = block_dot(w0, 4 * (1 - zb), 4)
        g0p1 = rdma(acc0, rcv0, ss0, rs0, 4 * (1 - zb), 4, 0, 0, p ^ 4)
        g0p1.start()
        acc1[pl.ds(2 * (1 - yb), 2)] = block_dot(w1, 2 * (1 - yb), 2)
        acc1[pl.ds(4 + 2 * (1 - yb), 2)] = block_dot(w1, 4 + 2 * (1 - yb), 2)
        g1p1a = rdma(acc1, rcv1, ss1, rs1, 2 * (1 - yb), 2, 0, 0, p ^ 3)
        g1p1b = rdma(acc1, rcv1, ss1, rs1, 4 + 2 * (1 - yb), 2, 2, 1, p ^ 3)
        g1p1a.start()
        g1p1b.start()
        sx_send = side_x(1 - xb)
        sx_keep = side_x(xb)
        for c in sx_send[:2]:
            acc2[pl.ds(c, 1)] = block_dot(w2, c, 1)
        g2p1 = [rdma(acc2, rcv2, ss2, rs2, sx_send[k], 1, k, k, p ^ 1)
                for k in range(2)]
        g2p1[0].start()
        g2p1[1].start()
        for c in sx_send[2:]:
            acc2[pl.ds(c, 1)] = block_dot(w2, c, 1)

        acc0[pl.ds(4 * zb, 4)] = block_dot(w0, 4 * zb, 4)
        acc1[pl.ds(2 * yb, 2)] = block_dot(w1, 2 * yb, 2)
        acc1[pl.ds(4 + 2 * yb, 2)] = block_dot(w1, 4 + 2 * yb, 2)
        for c in sx_keep:
            acc2[pl.ds(c, 1)] = block_dot(w2, c, 1)

        g2p1[0].wait()
        acc2[pl.ds(sx_keep[0], 1)] = acc2[pl.ds(sx_keep[0], 1)] + rcv2[pl.ds(0, 1)]
        g2p1.append(rdma(acc2, rcv2, ss2, rs2, sx_send[2], 1, 2, 2, p ^ 1))
        g2p1[2].start()
        g2p1[1].wait()
        acc2[pl.ds(sx_keep[1], 1)] = acc2[pl.ds(sx_keep[1], 1)] + rcv2[pl.ds(1, 1)]
        g2p1.append(rdma(acc2, rcv2, ss2, rs2, sx_send[3], 1, 3, 3, p ^ 1))
        g2p1[3].start()

        g1p1a.wait()
        acc1[pl.ds(2 * yb, 2)] = acc1[pl.ds(2 * yb, 2)] + rcv1[pl.ds(0, 2)]

        g2p1[2].wait()
        acc2[pl.ds(sx_keep[2], 1)] = acc2[pl.ds(sx_keep[2], 1)] + rcv2[pl.ds(2, 1)]
        g2p1[3].wait()
        acc2[pl.ds(sx_keep[3], 1)] = acc2[pl.ds(sx_keep[3], 1)] + rcv2[pl.ds(3, 1)]

        g0p1.wait()
        acc0[pl.ds(4 * zb, 4)] = acc0[pl.ds(4 * zb, 4)] + rcv0[pl.ds(0, 4)]
        g0p2 = rdma(acc0, rcv0, ss0, rs0, 4 * zb + 2 * (1 - yb), 2, 4, 1, p ^ 3)
        g0p2.start()

        g2s = [jnp.minimum(p ^ 4, p ^ 7), jnp.maximum(p ^ 4, p ^ 7)]
        g2k = [jnp.minimum(p, p ^ 3), jnp.maximum(p, p ^ 3)]
        g2p2 = [rdma(acc2, rcv2, ss2, rs2, g2s[k], 1, 4 + k, 4 + k, p ^ 4)
                for k in range(2)]
        g2p2[0].start()
        g2p2[1].start()

        g1p1b.wait()
        acc1[pl.ds(4 + 2 * yb, 2)] = acc1[pl.ds(4 + 2 * yb, 2)] + rcv1[pl.ds(2, 2)]
        q = (p ^ 1) & 3
        kk = p & 3
        g1p2 = [rdma(acc1, rcv1, ss1, rs1, q, 1, 4, 2, p ^ 1),
                rdma(acc1, rcv1, ss1, rs1, q + 4, 1, 5, 3, p ^ 1)]
        g1p2[0].start()
        g1p2[1].start()

        g0p2.wait()
        a0 = 4 * zb + 2 * yb
        acc0[pl.ds(a0, 2)] = acc0[pl.ds(a0, 2)] + rcv0[pl.ds(4, 2)]
        g0p3 = rdma(acc0, rcv0, ss0, rs0, p ^ 1, 1, 6, 2, p ^ 1)
        g0p3.start()

        g2p2[0].wait()
        acc2[pl.ds(g2k[0], 1)] = acc2[pl.ds(g2k[0], 1)] + rcv2[pl.ds(4, 1)]
        g2p2[1].wait()
        acc2[pl.ds(g2k[1], 1)] = acc2[pl.ds(g2k[1], 1)] + rcv2[pl.ds(5, 1)]
        g2p3 = rdma(acc2, rcv2, ss2, rs2, p ^ 3, 1, 6, 6, p ^ 3)
        g2p3.start()

        g1p2[0].wait()
        acc1[pl.ds(kk, 1)] = acc1[pl.ds(kk, 1)] + rcv1[pl.ds(4, 1)]
        g1p2[1].wait()
        acc1[pl.ds(kk + 4, 1)] = acc1[pl.ds(kk + 4, 1)] + rcv1[pl.ds(5, 1)]
        g1p3 = rdma(acc1, rcv1, ss1, rs1, p ^ 4, 1, 6, 4, p ^ 4)
        g1p3.start()

        scale = sx_ref[0] * sw_ref[0]
        g2p3.wait()
        out_ref[:, G2[0]:G2[1]] = (
            (acc2[p] + rcv2[6]).astype(jnp.float32) * scale)
        g0p3.wait()
        out_ref[:, G0[0]:G0[1]] = (
            (acc0[p] + rcv0[6]).astype(jnp.float32) * scale)
        g1p3.wait()
        out_ref[:, G1[0]:G1[1]] = (
            (acc1[p] + rcv1[6]).astype(jnp.float32) * scale)

    scratch = [
        pltpu.VMEM((N_DEV, M_PER, G0[1] - G0[0]), jnp.bfloat16),
        pltpu.VMEM((N_DEV, M_PER, G1[1] - G1[0]), jnp.bfloat16),
        pltpu.VMEM((N_DEV, M_PER, G2[1] - G2[0]), jnp.bfloat16),
        pltpu.VMEM((7, M_PER, G0[1] - G0[0]), jnp.bfloat16),
        pltpu.VMEM((7, M_PER, G1[1] - G1[0]), jnp.bfloat16),
        pltpu.VMEM((7, M_PER, G2[1] - G2[0]), jnp.bfloat16),
        pltpu.SemaphoreType.DMA((3,)),
        pltpu.SemaphoreType.DMA((3,)),
        pltpu.SemaphoreType.DMA((5,)),
        pltpu.SemaphoreType.DMA((5,)),
        pltpu.SemaphoreType.DMA((7,)),
        pltpu.SemaphoreType.DMA((7,)),
    ]

    return pl.pallas_call(
        body,
        out_shape=jax.ShapeDtypeStruct((M_PER, N), jnp.float32),
        in_specs=[
            pl.BlockSpec(memory_space=pltpu.VMEM),
            pl.BlockSpec(memory_space=pltpu.VMEM),
            pl.BlockSpec(memory_space=pltpu.SMEM),
            pl.BlockSpec(memory_space=pltpu.SMEM),
        ],
        out_specs=pl.BlockSpec(memory_space=pltpu.VMEM),
        scratch_shapes=scratch,
        compiler_params=pltpu.CompilerParams(
            collective_id=0,
            vmem_limit_bytes=100 * 1024 * 1024,
        ),
    )(x, w_mat, scale_x, scale_w)


# device time: 79206 ns/iter; 1.0505x vs baseline; 1.0505x over previous
import jax
import jax.numpy as jnp
from jax import lax
from jax.experimental import pallas as pl
from jax.experimental.pallas import tpu as pltpu

N_DEV = 8
M = 4096
M_PER = 512
N = 2048

GROUPS = (
    (0, 640, "zyx"),
    (640, 1408, "yxz"),
    (1408, 2048, "xzy"),
)


def _chunk_dot(x_ref, w_g, c):
    xs = x_ref[pl.ds(c * M_PER, M_PER), :]
    return lax.dot_general(
        xs, w_g, (((1,), (0,)), ((), ())),
        preferred_element_type=jnp.int32,
    ).astype(jnp.bfloat16)


def kernel(x, w_mat, scale_x, scale_w):
    n_grp = len(GROUPS)

    def body(x_ref, w_ref, sx_ref, sw_ref, out_ref, *scr):
        accs = scr[0:n_grp]
        rcvs = scr[n_grp:2 * n_grp]
        ssems = scr[2 * n_grp:3 * n_grp]
        rsems = scr[3 * n_grp:4 * n_grp]

        p = lax.axis_index("i")
        zb = p >> 2
        yb = (p >> 1) & 1
        xb = (p ^ (p >> 1)) & 1

        def side_x(t):
            return [jnp.where(t == 0, a, b)
                    for a, b in zip((0, 3, 4, 7), (1, 2, 5, 6))]

        def side_y(t):
            return [2 * t, 2 * t + 1, 4 + 2 * t, 4 + 2 * t + 1]

        def side_z(t):
            return [4 * t + j for j in range(4)]

        dims = {
            "x": (p ^ 1, xb, side_x),
            "y": (p ^ 3, yb, side_y),
            "z": (p ^ 4, zb, side_z),
        }

        def plan(order):
            d1, s1, f1 = dims[order[0]]
            d2, s2, f2 = dims[order[1]]
            d3, s3, _ = dims[order[2]]
            keep1 = f1(s1)
            send1 = f1(1 - s1)
            keep2 = _intersect(keep1, f2(s2))
            send2 = _intersect(keep1, f2(1 - s2))
            p3_send = [p ^ {"x": 1, "y": 3, "z": 4}[order[2]]]
            p3_keep = [p]
            return [(d1, send1, keep1), (d2, send2, keep2),
                    (d3, p3_send, p3_keep)]

        def _intersect(big, four):
            out = []
            for c in big:
                hit = jnp.zeros((), jnp.int32)
                for d in four:
                    hit = hit | jnp.where(c == d, 1, 0)
                out.append((c, hit))
            n_keep = len(big) // 2
            sel = []
            for k in range(n_keep):
                acc_idx = jnp.zeros((), jnp.int32)
                prefix = jnp.zeros((), jnp.int32)
                for c, hit in out:
                    take = (hit == 1) & (prefix == k)
                    acc_idx = jnp.where(take, c, acc_idx)
                    prefix = prefix + hit
                sel.append(acc_idx)
            return sel

        plans = [plan(order) for (_c0, _c1, order) in GROUPS]
        w_gs = [w_ref[:, c0:c1] for (c0, c1, _o) in GROUPS]
        slot0 = (0, 4, 6)

        barrier_sem = pltpu.get_barrier_semaphore()
        for nbr in (p ^ 1, p ^ 3, p ^ 4):
            pl.semaphore_signal(
                barrier_sem, inc=1,
                device_id=(nbr,), device_id_type=pl.DeviceIdType.MESH,
            )
        pl.semaphore_wait(barrier_sem, 3)

        def start_chunks(g, ph, ks):
            partner, send, _keep = plans[g][ph]
            out = []
            for k in ks:
                slot = slot0[ph] + k
                r = pltpu.make_async_remote_copy(
                    src_ref=accs[g].at[send[k]],
                    dst_ref=rcvs[g].at[slot],
                    send_sem=ssems[g].at[slot],
                    recv_sem=rsems[g].at[slot],
                    device_id=(partner,),
                    device_id_type=pl.DeviceIdType.MESH,
                )
                r.start()
                out.append(r)
            return out

        def finish_chunks(g, ph, ks, rdmas):
            _partner, _send, keep = plans[g][ph]
            for k, r in zip(ks, rdmas):
                r.wait()
                c = keep[k]
                accs[g][c] = accs[g][c] + rcvs[g][slot0[ph] + k]

        rd = [None] * n_grp
        for g in range(n_grp):
            for c in plans[g][0][1][:2]:
                accs[g][c] = _chunk_dot(x_ref, w_gs[g], c)
            rd[g] = start_chunks(g, 0, [0, 1])
        for g in range(n_grp):
            for c in plans[g][0][1][2:]:
                accs[g][c] = _chunk_dot(x_ref, w_gs[g], c)
        for g in range(n_grp):
            for c in plans[g][0][2]:
                accs[g][c] = _chunk_dot(x_ref, w_gs[g], c)

        for k in range(4):
            for g in range(n_grp):
                finish_chunks(g, 0, [k], [rd[g][k]])
                if k + 2 < 4:
                    rd[g].extend(start_chunks(g, 0, [k + 2]))
        for g in range(n_grp):
            rd[g] = start_chunks(g, 1, [0, 1])
        for g in range(n_grp):
            finish_chunks(g, 1, [0, 1], rd[g])
            rd[g] = start_chunks(g, 2, [0])

        scale = sx_ref[0] * sw_ref[0]
        for g in range(n_grp):
            rd[g][0].wait()
            c0, c1, _o = GROUPS[g]
            final = accs[g][p] + rcvs[g][slot0[2]]
            out_ref[:, c0:c1] = final.astype(jnp.float32) * scale

    scratch = []
    for c0, c1, _o in GROUPS:
        scratch.append(pltpu.VMEM((N_DEV, M_PER, c1 - c0), jnp.bfloat16))
    for c0, c1, _o in GROUPS:
        scratch.append(pltpu.VMEM((7, M_PER, c1 - c0), jnp.bfloat16))
    for _ in GROUPS:
        scratch.append(pltpu.SemaphoreType.DMA((7,)))
    for _ in GROUPS:
        scratch.append(pltpu.SemaphoreType.DMA((7,)))

    return pl.pallas_call(
        body,
        out_shape=jax.ShapeDtypeStruct((M_PER, N), jnp.float32),
        in_specs=[
            pl.BlockSpec(memory_space=pltpu.VMEM),
            pl.BlockSpec(memory_space=pltpu.VMEM),
            pl.BlockSpec(memory_space=pltpu.SMEM),
            pl.BlockSpec(memory_space=pltpu.SMEM),
        ],
        out_specs=pl.BlockSpec(memory_space=pltpu.VMEM),
        scratch_shapes=scratch,
        compiler_params=pltpu.CompilerParams(
            collective_id=0,
            vmem_limit_bytes=100 * 1024 * 1024,
        ),
    )(x, w_mat, scale_x, scale_w)
